# baseline (speedup 1.0000x reference)
import functools
import jax
import jax.numpy as jnp
from jax import lax
from jax.experimental import pallas as pl
from jax.experimental.pallas import tpu as pltpu
from jax.experimental.pallas import tpu_sc as plsc

_B = 16384
_D = 64
_L = 16
_NC = 2
_NS = 16
_NW = _NC * _NS
_BPW = _B // _NW

_mesh = plsc.VectorSubcoreMesh(core_axis_name="c", subcore_axis_name="s")


@functools.partial(
    pl.kernel,
    out_type=jax.ShapeDtypeStruct((_B, _D), jnp.float32),
    mesh=_mesh,
    compiler_params=pltpu.CompilerParams(use_tc_tiling_on_sc=False),
    scratch_types=[
        pltpu.VMEM((_BPW,), jnp.int32),
        pltpu.VMEM((_BPW,), jnp.int32),
        pltpu.VMEM((_BPW, _D), jnp.float32),
        pltpu.VMEM((_BPW, _D), jnp.float32),
        pltpu.SemaphoreType.DMA,
        pltpu.SemaphoreType.DMA,
    ],
)
def _emb_sum(x0_hbm, x1_hbm, w0_hbm, w1_hbm, out_hbm,
             idx0_v, idx1_v, rows0_v, rows1_v, sem0, sem1):
    wid = lax.axis_index("s") * _NC + lax.axis_index("c")
    base = wid * _BPW
    pltpu.sync_copy(x0_hbm.at[pl.ds(base, _BPW)], idx0_v)
    pltpu.sync_copy(x1_hbm.at[pl.ds(base, _BPW)], idx1_v)
    c0 = pltpu.async_copy(w0_hbm.at[idx0_v], rows0_v, sem0)
    c1 = pltpu.async_copy(w1_hbm.at[idx1_v], rows1_v, sem1)
    c0.wait()
    c1.wait()

    def body(i, carry):
        for j in range(_D // _L):
            sl = pl.ds(j * _L, _L)
            rows0_v[i, sl] = rows0_v[i, sl] + rows1_v[i, sl]
        return carry

    lax.fori_loop(0, _BPW, body, 0)
    pltpu.sync_copy(rows0_v, out_hbm.at[pl.ds(base, _BPW)])


def kernel(x, W0, W1):
    x = x.astype(jnp.int32)
    return _emb_sum(x[:, 0], x[:, 1], W0, W1)


# speedup vs baseline: 1.6967x; 1.6967x over previous
import functools
import jax
import jax.numpy as jnp
from jax import lax
from jax.experimental import pallas as pl
from jax.experimental.pallas import tpu as pltpu
from jax.experimental.pallas import tpu_sc as plsc

_V = 1000001
_B = 16384
_D = 64
_L = 16
_NC = 2
_NS = 16
_NW = _NC * _NS
_BPW = _B // _NW
_NCB = 7812
_VMAIN = _NCB * 128

_mesh = plsc.VectorSubcoreMesh(core_axis_name="c", subcore_axis_name="s")


@functools.partial(
    pl.kernel,
    out_type=jax.ShapeDtypeStruct((_D, _B), jnp.float32),
    mesh=_mesh,
    compiler_params=pltpu.CompilerParams(needs_layout_passes=False),
    scratch_types=[
        pltpu.VMEM((_BPW,), jnp.int32),
        pltpu.VMEM((_BPW,), jnp.int32),
        pltpu.VMEM((_D, 128), jnp.float32),
        pltpu.VMEM((_D, 128), jnp.float32),
        pltpu.VMEM((_D, 128), jnp.float32),
        pltpu.VMEM((_D, 128), jnp.float32),
        pltpu.VMEM((_D, _BPW), jnp.float32),
        pltpu.SemaphoreType.DMA,
        pltpu.SemaphoreType.DMA,
    ],
)
def _emb_sum(x0_hbm, x1_hbm, w0t_hbm, w1t_hbm, out_hbm,
             idx0_v, idx1_v, blk0a_v, blk0b_v, blk1a_v, blk1b_v,
             slab_v, sem0, sem1):
    wid = lax.axis_index("s") * _NC + lax.axis_index("c")
    base = wid * _BPW
    pltpu.sync_copy(x0_hbm.at[pl.ds(base, _BPW)], idx0_v)
    pltpu.sync_copy(x1_hbm.at[pl.ds(base, _BPW)], idx1_v)

    iota = lax.iota(jnp.int32, _L)
    dvs = [16 * c + iota for c in range(_D // _L)]
    blk0 = [blk0a_v, blk0b_v]
    blk1 = [blk1a_v, blk1b_v]

    def group(g, carry):
        r0 = g * _L
        vec0 = idx0_v[pl.ds(r0, _L)]
        vec1 = idx1_v[pl.ds(r0, _L)]
        cb0 = jnp.minimum(lax.shift_right_logical(vec0, 7), _NCB - 1)
        cb1 = jnp.minimum(lax.shift_right_logical(vec1, 7), _NCB - 1)
        lane0 = vec0 - cb0 * 128
        lane1 = vec1 - cb1 * 128

        cps = [None, None]

        def fire(j):
            p = j % 2
            cp0 = pltpu.async_copy(w0t_hbm.at[:, pl.ds(cb0[j] * 128, 128)],
                                   blk0[p], sem0)
            cp1 = pltpu.async_copy(w1t_hbm.at[:, pl.ds(cb1[j] * 128, 128)],
                                   blk1[p], sem1)
            cps[p] = (cp0, cp1)

        fire(0)
        for j in range(_L):
            p = j % 2
            cp0, cp1 = cps[p]
            cp0.wait()
            cp1.wait()
            if j + 1 < _L:
                fire(j + 1)
            l0 = lane0[j]
            l1 = lane1[j]
            f0 = jnp.full((_L,), (l0 < 128).astype(jnp.float32))
            f1 = jnp.full((_L,), (l1 < 128).astype(jnp.float32))
            l0v = jnp.full((_L,), jnp.minimum(l0, 127))
            l1v = jnp.full((_L,), jnp.minimum(l1, 127))
            rv = jnp.full((_L,), r0 + j)
            for c in range(_D // _L):
                e0 = plsc.load_gather(blk0[p], [dvs[c], l0v])
                e1 = plsc.load_gather(blk1[p], [dvs[c], l1v])
                plsc.store_scatter(slab_v, [dvs[c], rv], e0 * f0 + e1 * f1)
        return carry

    lax.fori_loop(0, _BPW // _L, group, 0)
    pltpu.sync_copy(slab_v, out_hbm.at[:, pl.ds(base, _BPW)])


def kernel(x, W0, W1):
    x = x.astype(jnp.int32)
    x0 = x[:, 0]
    x1 = x[:, 1]
    out = _emb_sum(x0, x1, W0.T, W1.T).T
    tail0 = jnp.take(W0[_VMAIN:], jnp.clip(x0 - _VMAIN, 0, _V - _VMAIN - 1),
                     axis=0)
    tail1 = jnp.take(W1[_VMAIN:], jnp.clip(x1 - _VMAIN, 0, _V - _VMAIN - 1),
                     axis=0)
    out = out + jnp.where((x0 >= _VMAIN)[:, None], tail0, 0.0)
    out = out + jnp.where((x1 >= _VMAIN)[:, None], tail1, 0.0)
    return out


# speedup vs baseline: 2.5824x; 1.5220x over previous
import functools
import jax
import jax.numpy as jnp
from jax import lax
from jax.experimental import pallas as pl
from jax.experimental.pallas import tpu as pltpu
from jax.experimental.pallas import tpu_sc as plsc

_V = 1000001
_B = 16384
_D = 64
_L = 16
_NC = 2
_NS = 16
_NW = _NC * _NS
_BPW = _B // _NW
_NCB = 7812
_VMAIN = _NCB * 128

_mesh = plsc.VectorSubcoreMesh(core_axis_name="c", subcore_axis_name="s")


@functools.partial(
    pl.kernel,
    out_type=jax.ShapeDtypeStruct((_D, _B), jnp.float32),
    mesh=_mesh,
    compiler_params=pltpu.CompilerParams(needs_layout_passes=False),
    scratch_types=[
        pltpu.VMEM((_BPW,), jnp.int32),
        pltpu.VMEM((_BPW,), jnp.int32),
        pltpu.VMEM((_D, 128), jnp.float32),
        pltpu.VMEM((_D, 128), jnp.float32),
        pltpu.VMEM((_D, 128), jnp.float32),
        pltpu.VMEM((_D, 128), jnp.float32),
        pltpu.VMEM((_D, 128), jnp.float32),
        pltpu.VMEM((_D, 128), jnp.float32),
        pltpu.VMEM((_D, 128), jnp.float32),
        pltpu.VMEM((_D, 128), jnp.float32),
        pltpu.VMEM((_D, _BPW), jnp.float32),
        pltpu.SemaphoreType.DMA,
        pltpu.SemaphoreType.DMA,
    ],
)
def _emb_sum(x0_hbm, x1_hbm, w0t_hbm, w1t_hbm, out_hbm,
             idx0_v, idx1_v, blk0a_v, blk0b_v, blk0c_v, blk0d_v,
             blk1a_v, blk1b_v, blk1c_v, blk1d_v,
             slab_v, sem0, sem1):
    wid = lax.axis_index("s") * _NC + lax.axis_index("c")
    base = wid * _BPW
    pltpu.sync_copy(x0_hbm.at[pl.ds(base, _BPW)], idx0_v)
    pltpu.sync_copy(x1_hbm.at[pl.ds(base, _BPW)], idx1_v)

    iota = lax.iota(jnp.int32, _L)
    dvs = [16 * c + iota for c in range(_D // _L)]
    blk0 = [blk0a_v, blk0b_v, blk0c_v, blk0d_v]
    blk1 = [blk1a_v, blk1b_v, blk1c_v, blk1d_v]
    _RING = 4

    def group(g, carry):
        r0 = g * _L
        vec0 = idx0_v[pl.ds(r0, _L)]
        vec1 = idx1_v[pl.ds(r0, _L)]
        cb0 = jnp.minimum(lax.shift_right_logical(vec0, 7), _NCB - 1)
        cb1 = jnp.minimum(lax.shift_right_logical(vec1, 7), _NCB - 1)
        lane0 = vec0 - cb0 * 128
        lane1 = vec1 - cb1 * 128

        cps = [None] * _RING

        def fire(j):
            p = j % _RING
            cp0 = pltpu.async_copy(w0t_hbm.at[:, pl.ds(cb0[j] * 128, 128)],
                                   blk0[p], sem0)
            cp1 = pltpu.async_copy(w1t_hbm.at[:, pl.ds(cb1[j] * 128, 128)],
                                   blk1[p], sem1)
            cps[p] = (cp0, cp1)

        for j in range(_RING - 1):
            fire(j)
        for j in range(_L):
            p = j % _RING
            cp0, cp1 = cps[p]
            cp0.wait()
            cp1.wait()
            if j + _RING - 1 < _L:
                fire(j + _RING - 1)
            l0 = lane0[j]
            l1 = lane1[j]
            f0 = jnp.full((_L,), (l0 < 128).astype(jnp.float32))
            f1 = jnp.full((_L,), (l1 < 128).astype(jnp.float32))
            l0v = jnp.full((_L,), jnp.minimum(l0, 127))
            l1v = jnp.full((_L,), jnp.minimum(l1, 127))
            rv = jnp.full((_L,), r0 + j)
            for c in range(_D // _L):
                e0 = plsc.load_gather(blk0[p], [dvs[c], l0v])
                e1 = plsc.load_gather(blk1[p], [dvs[c], l1v])
                plsc.store_scatter(slab_v, [dvs[c], rv], e0 * f0 + e1 * f1)
        return carry

    lax.fori_loop(0, _BPW // _L, group, 0)
    pltpu.sync_copy(slab_v, out_hbm.at[:, pl.ds(base, _BPW)])


def kernel(x, W0, W1):
    x = x.astype(jnp.int32)
    x0 = x[:, 0]
    x1 = x[:, 1]
    out = _emb_sum(x0, x1, W0.T, W1.T).T
    tail0 = jnp.take(W0[_VMAIN:], jnp.clip(x0 - _VMAIN, 0, _V - _VMAIN - 1),
                     axis=0)
    tail1 = jnp.take(W1[_VMAIN:], jnp.clip(x1 - _VMAIN, 0, _V - _VMAIN - 1),
                     axis=0)
    out = out + jnp.where((x0 >= _VMAIN)[:, None], tail0, 0.0)
    out = out + jnp.where((x1 >= _VMAIN)[:, None], tail1, 0.0)
    return out


# speedup vs baseline: 2.7268x; 1.0559x over previous
import functools
import jax
import jax.numpy as jnp
from jax import lax
from jax.experimental import pallas as pl
from jax.experimental.pallas import tpu as pltpu
from jax.experimental.pallas import tpu_sc as plsc

_V = 1000001
_B = 16384
_D = 64
_L = 16
_NC = 2
_NS = 16
_NW = _NC * _NS
_BPW = _B // _NW
_NCB = 7812
_VMAIN = _NCB * 128

_mesh = plsc.VectorSubcoreMesh(core_axis_name="c", subcore_axis_name="s")


@functools.partial(
    pl.kernel,
    out_type=jax.ShapeDtypeStruct((_D, _B), jnp.float32),
    mesh=_mesh,
    compiler_params=pltpu.CompilerParams(needs_layout_passes=False),
    scratch_types=[
        pltpu.VMEM((_BPW,), jnp.int32),
        pltpu.VMEM((_BPW,), jnp.int32),
        pltpu.VMEM((_D, 128), jnp.float32),
        pltpu.VMEM((_D, 128), jnp.float32),
        pltpu.VMEM((_D, 128), jnp.float32),
        pltpu.VMEM((_D, 128), jnp.float32),
        pltpu.VMEM((_D, 128), jnp.float32),
        pltpu.VMEM((_D, 128), jnp.float32),
        pltpu.VMEM((_D, 128), jnp.float32),
        pltpu.VMEM((_D, 128), jnp.float32),
        pltpu.VMEM((_D, 128), jnp.float32),
        pltpu.VMEM((_D, 128), jnp.float32),
        pltpu.VMEM((_D, _BPW), jnp.float32),
        pltpu.SemaphoreType.DMA,
        pltpu.SemaphoreType.DMA,
    ],
)
def _emb_sum(x0_hbm, x1_hbm, w0t_hbm, w1t_hbm, out_hbm,
             idx0_v, idx1_v, blk0a_v, blk0b_v, blk0c_v, blk0d_v, blk0e_v,
             blk1a_v, blk1b_v, blk1c_v, blk1d_v, blk1e_v,
             slab_v, sem0, sem1):
    wid = lax.axis_index("s") * _NC + lax.axis_index("c")
    base = wid * _BPW
    pltpu.sync_copy(x0_hbm.at[pl.ds(base, _BPW)], idx0_v)
    pltpu.sync_copy(x1_hbm.at[pl.ds(base, _BPW)], idx1_v)

    iota = lax.iota(jnp.int32, _L)
    dvs = [16 * c + iota for c in range(_D // _L)]
    blk0 = [blk0a_v, blk0b_v, blk0c_v, blk0d_v, blk0e_v]
    blk1 = [blk1a_v, blk1b_v, blk1c_v, blk1d_v, blk1e_v]
    _RING = 5

    _G = 2 * _L

    def group(g, carry):
        r0 = g * _G
        vecs0 = [idx0_v[pl.ds(r0, _L)], idx0_v[pl.ds(r0 + _L, _L)]]
        vecs1 = [idx1_v[pl.ds(r0, _L)], idx1_v[pl.ds(r0 + _L, _L)]]
        cbs0 = [jnp.minimum(lax.shift_right_logical(v, 7), _NCB - 1)
                for v in vecs0]
        cbs1 = [jnp.minimum(lax.shift_right_logical(v, 7), _NCB - 1)
                for v in vecs1]
        lanes0 = [v - cb * 128 for v, cb in zip(vecs0, cbs0)]
        lanes1 = [v - cb * 128 for v, cb in zip(vecs1, cbs1)]

        cps = [None] * _RING

        def fire(j):
            p = j % _RING
            q, r = divmod(j, _L)
            cp0 = pltpu.async_copy(
                w0t_hbm.at[:, pl.ds(cbs0[q][r] * 128, 128)], blk0[p], sem0)
            cp1 = pltpu.async_copy(
                w1t_hbm.at[:, pl.ds(cbs1[q][r] * 128, 128)], blk1[p], sem1)
            cps[p] = (cp0, cp1)

        for j in range(_RING - 1):
            fire(j)
        for j in range(_G):
            p = j % _RING
            cp0, cp1 = cps[p]
            cp0.wait()
            cp1.wait()
            if j + _RING - 1 < _G:
                fire(j + _RING - 1)
            q, r = divmod(j, _L)
            l0 = lanes0[q][r]
            l1 = lanes1[q][r]
            f0 = jnp.full((_L,), (l0 < 128).astype(jnp.float32))
            f1 = jnp.full((_L,), (l1 < 128).astype(jnp.float32))
            l0v = jnp.full((_L,), jnp.minimum(l0, 127))
            l1v = jnp.full((_L,), jnp.minimum(l1, 127))
            rv = jnp.full((_L,), r0 + j)
            for c in range(_D // _L):
                e0 = plsc.load_gather(blk0[p], [dvs[c], l0v])
                e1 = plsc.load_gather(blk1[p], [dvs[c], l1v])
                plsc.store_scatter(slab_v, [dvs[c], rv], e0 * f0 + e1 * f1)
        return carry

    lax.fori_loop(0, _BPW // _G, group, 0)
    pltpu.sync_copy(slab_v, out_hbm.at[:, pl.ds(base, _BPW)])


def kernel(x, W0, W1):
    x = x.astype(jnp.int32)
    x0 = x[:, 0]
    x1 = x[:, 1]
    out = _emb_sum(x0, x1, W0.T, W1.T).T
    tail0 = jnp.take(W0[_VMAIN:], jnp.clip(x0 - _VMAIN, 0, _V - _VMAIN - 1),
                     axis=0)
    tail1 = jnp.take(W1[_VMAIN:], jnp.clip(x1 - _VMAIN, 0, _V - _VMAIN - 1),
                     axis=0)
    out = out + jnp.where((x0 >= _VMAIN)[:, None], tail0, 0.0)
    out = out + jnp.where((x1 >= _VMAIN)[:, None], tail1, 0.0)
    return out


# speedup vs baseline: 2.7830x; 1.0206x over previous
import functools
import jax
import jax.numpy as jnp
from jax import lax
from jax.experimental import pallas as pl
from jax.experimental.pallas import tpu as pltpu
from jax.experimental.pallas import tpu_sc as plsc

_V = 1000001
_B = 16384
_D = 64
_L = 16
_NC = 2
_NS = 16
_NW = _NC * _NS
_BPW = _B // _NW
_NCB = 7812
_VMAIN = _NCB * 128

_mesh = plsc.VectorSubcoreMesh(core_axis_name="c", subcore_axis_name="s")


@functools.partial(
    pl.kernel,
    out_type=jax.ShapeDtypeStruct((_D, _B), jnp.float32),
    mesh=_mesh,
    compiler_params=pltpu.CompilerParams(needs_layout_passes=False),
    scratch_types=[
        pltpu.VMEM((_BPW,), jnp.int32),
        pltpu.VMEM((_BPW,), jnp.int32),
        pltpu.VMEM((_D, 128), jnp.float32),
        pltpu.VMEM((_D, 128), jnp.float32),
        pltpu.VMEM((_D, 128), jnp.float32),
        pltpu.VMEM((_D, 128), jnp.float32),
        pltpu.VMEM((_D, 128), jnp.float32),
        pltpu.VMEM((_D, 128), jnp.float32),
        pltpu.VMEM((_D, 128), jnp.float32),
        pltpu.VMEM((_D, 128), jnp.float32),
        pltpu.VMEM((_D, 128), jnp.float32),
        pltpu.VMEM((_D, 128), jnp.float32),
        pltpu.VMEM((_D, 128), jnp.float32),
        pltpu.VMEM((_D, 128), jnp.float32),
        pltpu.VMEM((_D, _BPW // 2), jnp.float32),
        pltpu.SemaphoreType.DMA,
        pltpu.SemaphoreType.DMA,
    ],
)
def _emb_sum(x0_hbm, x1_hbm, w0t_hbm, w1t_hbm, out_hbm,
             idx0_v, idx1_v,
             blk0a_v, blk0b_v, blk0c_v, blk0d_v, blk0e_v, blk0f_v,
             blk1a_v, blk1b_v, blk1c_v, blk1d_v, blk1e_v, blk1f_v,
             slab_v, sem0, sem1):
    wid = lax.axis_index("s") * _NC + lax.axis_index("c")
    base = wid * _BPW
    pltpu.sync_copy(x0_hbm.at[pl.ds(base, _BPW)], idx0_v)
    pltpu.sync_copy(x1_hbm.at[pl.ds(base, _BPW)], idx1_v)

    iota = lax.iota(jnp.int32, _L)
    dvs = [16 * c + iota for c in range(_D // _L)]
    blk0 = [blk0a_v, blk0b_v, blk0c_v, blk0d_v, blk0e_v, blk0f_v]
    blk1 = [blk1a_v, blk1b_v, blk1c_v, blk1d_v, blk1e_v, blk1f_v]
    _RING = 6

    _G = 2 * _L

    def group(g, carry):
        r0 = g * _G
        vecs0 = [idx0_v[pl.ds(r0, _L)], idx0_v[pl.ds(r0 + _L, _L)]]
        vecs1 = [idx1_v[pl.ds(r0, _L)], idx1_v[pl.ds(r0 + _L, _L)]]
        cbs0 = [jnp.minimum(lax.shift_right_logical(v, 7), _NCB - 1)
                for v in vecs0]
        cbs1 = [jnp.minimum(lax.shift_right_logical(v, 7), _NCB - 1)
                for v in vecs1]
        lanes0 = [v - cb * 128 for v, cb in zip(vecs0, cbs0)]
        lanes1 = [v - cb * 128 for v, cb in zip(vecs1, cbs1)]

        cps = [None] * _RING

        def fire(j):
            p = j % _RING
            q, r = divmod(j, _L)
            cp0 = pltpu.async_copy(
                w0t_hbm.at[:, pl.ds(cbs0[q][r] * 128, 128)], blk0[p], sem0)
            cp1 = pltpu.async_copy(
                w1t_hbm.at[:, pl.ds(cbs1[q][r] * 128, 128)], blk1[p], sem1)
            cps[p] = (cp0, cp1)

        for j in range(_RING - 1):
            fire(j)
        for j in range(_G):
            p = j % _RING
            cp0, cp1 = cps[p]
            cp0.wait()
            cp1.wait()
            if j + _RING - 1 < _G:
                fire(j + _RING - 1)
            q, r = divmod(j, _L)
            l0 = lanes0[q][r]
            l1 = lanes1[q][r]
            f0 = jnp.full((_L,), (l0 < 128).astype(jnp.float32))
            f1 = jnp.full((_L,), (l1 < 128).astype(jnp.float32))
            l0v = jnp.full((_L,), jnp.minimum(l0, 127))
            l1v = jnp.full((_L,), jnp.minimum(l1, 127))
            rv = jnp.full((_L,), lax.rem(g, 8) * _G + j)
            for c in range(_D // _L):
                e0 = plsc.load_gather(blk0[p], [dvs[c], l0v])
                e1 = plsc.load_gather(blk1[p], [dvs[c], l1v])
                plsc.store_scatter(slab_v, [dvs[c], rv], e0 * f0 + e1 * f1)
        @pl.when(lax.rem(g, 8) == 7)
        def _flush():
            half = lax.div(g, 8) * (_BPW // 2)
            pltpu.sync_copy(slab_v, out_hbm.at[:, pl.ds(base + half, _BPW // 2)])

        return carry

    lax.fori_loop(0, _BPW // _G, group, 0)


def kernel(x, W0, W1):
    x = x.astype(jnp.int32)
    x0 = x[:, 0]
    x1 = x[:, 1]
    out = _emb_sum(x0, x1, W0.T, W1.T).T
    tail0 = jnp.take(W0[_VMAIN:], jnp.clip(x0 - _VMAIN, 0, _V - _VMAIN - 1),
                     axis=0)
    tail1 = jnp.take(W1[_VMAIN:], jnp.clip(x1 - _VMAIN, 0, _V - _VMAIN - 1),
                     axis=0)
    out = out + jnp.where((x0 >= _VMAIN)[:, None], tail0, 0.0)
    out = out + jnp.where((x1 >= _VMAIN)[:, None], tail1, 0.0)
    return out
